# Initial kernel scaffold; baseline (speedup 1.0000x reference)
import jax
import jax.numpy as jnp
from jax.experimental import pallas as pl


def kernel(x, edge_index, edge_weight, bn0_gamma, bn0_beta, bn1_gamma, bn1_beta, W_self0, W_neigh0, b0, W_self1, W_neigh1, b1, lin_W, lin_b):
    raise NotImplementedError("write your pallas kernel here")


# speedup vs baseline: 3.7214x; 3.7214x over previous
import functools

import jax
import jax.numpy as jnp
from jax import lax
from jax.experimental import pallas as pl
from jax.experimental.pallas import tpu as pltpu
from jax.experimental.pallas import tpu_sc as plsc

N = 10000
E = 320000
D = 128
NC = 2
NS = 16
LANES = 16
NW = NC * NS
EPW = E // NW
CHUNK = 80
NCHUNK = EPW // CHUNK
RPT = 624
RSPAN = 640


def _make_sc_agg():
  mesh = plsc.VectorSubcoreMesh(core_axis_name="c", subcore_axis_name="s")
  out_type = [jax.ShapeDtypeStruct((NC, N, D), jnp.float32)]
  scratch = [
      pltpu.VMEM((CHUNK,), jnp.int32),
      pltpu.VMEM((CHUNK,), jnp.int32),
      pltpu.VMEM((CHUNK,), jnp.float32),
      pltpu.VMEM((CHUNK, D), jnp.float32),
      pltpu.VMEM_SHARED((N, D), jnp.float32),
      pltpu.SemaphoreType.DMA,
  ]

  def body(h, src, dst, ew, out_agg, sidx, didx, ewv, rows, shagg, sem):
    c = lax.axis_index("c")
    s = lax.axis_index("s")
    row0 = s * RPT

    z16 = jnp.zeros((LANES,), jnp.float32)

    def zero_row(r, carry):
      for j in range(D // LANES):
        rows[r, pl.ds(j * LANES, LANES)] = z16
      return carry

    lax.fori_loop(0, CHUNK, zero_row, 0)
    for off in range(0, RSPAN, CHUNK):
      pltpu.sync_copy(rows, shagg.at[pl.ds(row0 + off, CHUNK)])
    plsc.subcore_barrier()

    ebase = (s * NC + c) * EPW

    def chunk_body(i, carry):
      base = ebase + i * CHUNK
      pltpu.sync_copy(src.at[pl.ds(base, CHUNK)], sidx)
      pltpu.sync_copy(dst.at[pl.ds(base, CHUNK)], didx)
      pltpu.sync_copy(ew.at[pl.ds(base, CHUNK)], ewv)
      pltpu.async_copy(h.at[sidx], rows, sem).wait()

      def scale_group(g, inner):
        w16 = ewv[pl.ds(g * LANES, LANES)]
        for k in range(LANES):
          w = lax.gather(
              w16, jnp.full((LANES, 1), k, jnp.int32),
              lax.GatherDimensionNumbers(offset_dims=(),
                                         collapsed_slice_dims=(0,),
                                         start_index_map=(0,)),
              (1,), mode=lax.GatherScatterMode.PROMISE_IN_BOUNDS)
          r = g * LANES + k
          for j in range(D // LANES):
            sl = pl.ds(j * LANES, LANES)
            rows[r, sl] = rows[r, sl] * w
        return inner

      lax.fori_loop(0, CHUNK // LANES, scale_group, 0)
      pltpu.sync_copy(rows, shagg.at[didx], add=True)
      return carry

    lax.fori_loop(0, NCHUNK, chunk_body, 0)
    plsc.subcore_barrier()

    pltpu.sync_copy(shagg.at[pl.ds(row0, RSPAN)],
                    out_agg.at[c, pl.ds(row0, RSPAN)])

  return pl.kernel(body, out_type, mesh=mesh, scratch_types=scratch)


def _make_sc_deg():
  mesh = plsc.VectorSubcoreMesh(core_axis_name="c", subcore_axis_name="s")
  out_type = [jax.ShapeDtypeStruct((NC, N, D), jnp.float32)]
  scratch = [
      pltpu.VMEM((CHUNK,), jnp.int32),
      pltpu.VMEM((CHUNK, D), jnp.float32),
      pltpu.VMEM_SHARED((N, D), jnp.float32),
  ]

  def body(dst, out_deg, didx, ones, shdeg):
    c = lax.axis_index("c")
    s = lax.axis_index("s")
    row0 = s * RPT

    z16 = jnp.zeros((LANES,), jnp.float32)

    def zero_row(r, carry):
      for j in range(D // LANES):
        ones[r, pl.ds(j * LANES, LANES)] = z16
      return carry

    lax.fori_loop(0, CHUNK, zero_row, 0)
    for off in range(0, RSPAN, CHUNK):
      pltpu.sync_copy(ones, shdeg.at[pl.ds(row0 + off, CHUNK)])

    o16 = jnp.ones((LANES,), jnp.float32)

    def one_row(r, carry):
      for j in range(D // LANES):
        ones[r, pl.ds(j * LANES, LANES)] = o16
      return carry

    lax.fori_loop(0, CHUNK, one_row, 0)
    plsc.subcore_barrier()

    ebase = (s * NC + c) * EPW

    def chunk_body(i, carry):
      base = ebase + i * CHUNK
      pltpu.sync_copy(dst.at[pl.ds(base, CHUNK)], didx)
      pltpu.sync_copy(ones, shdeg.at[didx], add=True)
      return carry

    lax.fori_loop(0, NCHUNK, chunk_body, 0)
    plsc.subcore_barrier()

    pltpu.sync_copy(shdeg.at[pl.ds(row0, RSPAN)],
                    out_deg.at[c, pl.ds(row0, RSPAN)])

  return pl.kernel(body, out_type, mesh=mesh, scratch_types=scratch)


_sc_agg = _make_sc_agg()
_sc_deg = _make_sc_deg()


def _tc_prep_body(x_ref, ew_ref, g_ref, b_ref, h_ref, ewl_ref):
  x = x_ref[...]
  mu = jnp.mean(x, axis=0, keepdims=True)
  var = jnp.mean((x - mu) ** 2, axis=0, keepdims=True)
  h_ref[...] = (x - mu) * lax.rsqrt(var + 1e-5) * g_ref[...] + b_ref[...]
  ewl_ref[...] = jnp.log1p(ew_ref[...])


def _tc_layer_body(h_ref, p_ref, deg_ref, g_ref, be_ref, ws_ref, wn_ref,
                   b_ref, out_ref, invdeg_ref):
  h = h_ref[...]
  agg = p_ref[0] + p_ref[1]
  deg = deg_ref[0] + deg_ref[1]
  inv = 1.0 / jnp.maximum(deg, 1.0)
  invdeg_ref[...] = inv
  z = (jnp.dot(h, ws_ref[...], preferred_element_type=jnp.float32)
       + jnp.dot(agg * inv, wn_ref[...], preferred_element_type=jnp.float32)
       + b_ref[...])
  z = jnp.maximum(z, 0.0)
  mu = jnp.mean(z, axis=0, keepdims=True)
  var = jnp.mean((z - mu) ** 2, axis=0, keepdims=True)
  out_ref[...] = (z - mu) * lax.rsqrt(var + 1e-5) * g_ref[...] + be_ref[...]


def _tc_final_body(h_ref, q_ref, invdeg_ref, ws_ref, wn_ref, b_ref,
                   lw_ref, lb_ref, out_ref):
  h = h_ref[...]
  neigh = (q_ref[0] + q_ref[1]) * invdeg_ref[...]
  z = (jnp.dot(h, ws_ref[...], preferred_element_type=jnp.float32)
       + jnp.dot(neigh, wn_ref[...], preferred_element_type=jnp.float32)
       + b_ref[...])
  z = jnp.maximum(z, 0.0)
  out_ref[...] = (jnp.dot(z, lw_ref[...], preferred_element_type=jnp.float32)
                  + lb_ref[...])


_tc_prep = pl.pallas_call(
    _tc_prep_body,
    out_shape=[jax.ShapeDtypeStruct((N, D), jnp.float32),
               jax.ShapeDtypeStruct((E // D, D), jnp.float32)],
)

_tc_layer = pl.pallas_call(
    _tc_layer_body,
    out_shape=[jax.ShapeDtypeStruct((N, D), jnp.float32),
               jax.ShapeDtypeStruct((N, 1), jnp.float32)],
)

_tc_final = pl.pallas_call(
    _tc_final_body,
    out_shape=jax.ShapeDtypeStruct((N, 1), jnp.float32),
)


def kernel(x, edge_index, edge_weight, bn0_gamma, bn0_beta, bn1_gamma,
           bn1_beta, W_self0, W_neigh0, b0, W_self1, W_neigh1, b1,
           lin_W, lin_b):
  src = edge_index[0]
  dst = edge_index[1]
  h0, ewl2d = _tc_prep(x, edge_weight.reshape(E // D, D),
                       bn0_gamma.reshape(1, D), bn0_beta.reshape(1, D))
  ewl = ewl2d.reshape(E)
  (degp,) = _sc_deg(dst)
  (p_agg,) = _sc_agg(h0, src, dst, ewl)
  h1n, invdeg = _tc_layer(h0, p_agg, degp[:, :, 0:1],
                          bn1_gamma.reshape(1, D),
                          bn1_beta.reshape(1, D), W_self0, W_neigh0,
                          b0.reshape(1, D))
  (q_agg,) = _sc_agg(h1n, src, dst, ewl)
  out = _tc_final(h1n, q_agg, invdeg, W_self1, W_neigh1, b1.reshape(1, D),
                  lin_W, lin_b.reshape(1, 1))
  return out[:, 0]


# speedup vs baseline: 6.6525x; 1.7876x over previous
import functools

import jax
import jax.numpy as jnp
from jax import lax
from jax.experimental import pallas as pl
from jax.experimental.pallas import tpu as pltpu
from jax.experimental.pallas import tpu_sc as plsc

N = 10000
E = 320000
D = 128
NC = 2
NS = 16
LANES = 16
NW = NC * NS
EPW = E // NW
CHUNK = 80
NCHUNK = EPW // CHUNK
RPT = 624
RSPAN = 640


NBUF = 3
SGN = 5
SGC = NCHUNK // SGN


def _make_sc_agg():
  mesh = plsc.VectorSubcoreMesh(core_axis_name="c", subcore_axis_name="s")
  out_type = [jax.ShapeDtypeStruct((NC, N, D), jnp.float32)]
  scratch = [
      pltpu.VMEM((SGC, CHUNK), jnp.int32),
      pltpu.VMEM((SGC, CHUNK), jnp.int32),
      pltpu.VMEM((SGC, CHUNK), jnp.float32),
      [pltpu.VMEM((CHUNK, D), jnp.float32) for _ in range(NBUF)],
      [pltpu.SemaphoreType.DMA for _ in range(NBUF)],
      [pltpu.SemaphoreType.DMA for _ in range(NBUF)],
      pltpu.VMEM_SHARED((N, D), jnp.float32),
  ]

  def body(h, src4, dst4, ew4, out_agg, sidx, didx, ewv, rows, gsems,
           ssems, shagg):
    c = lax.axis_index("c")
    s = lax.axis_index("s")
    wid = s * NC + c
    row0 = s * RPT

    z16 = jnp.zeros((LANES,), jnp.float32)

    def zero_row(r, carry):
      for j in range(D // LANES):
        rows[0][r, pl.ds(j * LANES, LANES)] = z16
      return carry

    lax.fori_loop(0, CHUNK, zero_row, 0)
    for off in range(0, RSPAN, CHUNK):
      pltpu.sync_copy(rows[0], shagg.at[pl.ds(row0 + off, CHUNK)])
    plsc.subcore_barrier()

    def scale_chunk(i, buf):
      def scale_group(g, inner):
        w16 = ewv[i, pl.ds(g * LANES, LANES)]
        for k in range(LANES):
          w = lax.gather(
              w16, jnp.full((LANES, 1), k, jnp.int32),
              lax.GatherDimensionNumbers(offset_dims=(),
                                         collapsed_slice_dims=(0,),
                                         start_index_map=(0,)),
              (1,), mode=lax.GatherScatterMode.PROMISE_IN_BOUNDS)
          r = g * LANES + k
          for j in range(D // LANES):
            sl = pl.ds(j * LANES, LANES)
            buf[r, sl] = buf[r, sl] * w
        return inner

      lax.fori_loop(0, CHUNK // LANES, scale_group, 0)

    def super_body(sg, carry):
      pltpu.sync_copy(src4.at[wid, sg], sidx)
      pltpu.sync_copy(dst4.at[wid, sg], didx)
      pltpu.sync_copy(ew4.at[wid, sg], ewv)

      def group_body(p, inner):
        i0 = p * NBUF
        gd = []
        for b in range(NBUF):
          gd.append(pltpu.async_copy(h.at[sidx.at[i0 + b]], rows[b],
                                     gsems[b]))
        sd = []
        for b in range(NBUF):
          gd[b].wait()
          scale_chunk(i0 + b, rows[b])
          sd.append(pltpu.async_copy(rows[b], shagg.at[didx.at[i0 + b]],
                                     ssems[b], add=True))
        for b in range(NBUF):
          sd[b].wait()
        return inner

      lax.fori_loop(0, SGC // NBUF, group_body, 0)

      last = SGC - 1
      pltpu.async_copy(h.at[sidx.at[last]], rows[0], gsems[0]).wait()
      scale_chunk(last, rows[0])
      pltpu.async_copy(rows[0], shagg.at[didx.at[last]], ssems[0],
                       add=True).wait()
      return carry

    lax.fori_loop(0, SGN, super_body, 0)

    plsc.subcore_barrier()
    pltpu.sync_copy(shagg.at[pl.ds(row0, RSPAN)],
                    out_agg.at[c, pl.ds(row0, RSPAN)])

  return pl.kernel(body, out_type, mesh=mesh, scratch_types=scratch)


def _make_sc_deg():
  mesh = plsc.VectorSubcoreMesh(core_axis_name="c", subcore_axis_name="s")
  out_type = [jax.ShapeDtypeStruct((NC, N, D), jnp.float32)]
  scratch = [
      pltpu.VMEM((CHUNK,), jnp.int32),
      pltpu.VMEM((CHUNK, D), jnp.float32),
      pltpu.VMEM_SHARED((N, D), jnp.float32),
  ]

  def body(dst, out_deg, didx, ones, shdeg):
    c = lax.axis_index("c")
    s = lax.axis_index("s")
    row0 = s * RPT

    z16 = jnp.zeros((LANES,), jnp.float32)

    def zero_row(r, carry):
      for j in range(D // LANES):
        ones[r, pl.ds(j * LANES, LANES)] = z16
      return carry

    lax.fori_loop(0, CHUNK, zero_row, 0)
    for off in range(0, RSPAN, CHUNK):
      pltpu.sync_copy(ones, shdeg.at[pl.ds(row0 + off, CHUNK)])

    o16 = jnp.ones((LANES,), jnp.float32)

    def one_row(r, carry):
      for j in range(D // LANES):
        ones[r, pl.ds(j * LANES, LANES)] = o16
      return carry

    lax.fori_loop(0, CHUNK, one_row, 0)
    plsc.subcore_barrier()

    ebase = (s * NC + c) * EPW

    def chunk_body(i, carry):
      base = ebase + i * CHUNK
      pltpu.sync_copy(dst.at[pl.ds(base, CHUNK)], didx)
      pltpu.sync_copy(ones, shdeg.at[didx], add=True)
      return carry

    lax.fori_loop(0, NCHUNK, chunk_body, 0)
    plsc.subcore_barrier()

    pltpu.sync_copy(shdeg.at[pl.ds(row0, RSPAN)],
                    out_deg.at[c, pl.ds(row0, RSPAN)])

  return pl.kernel(body, out_type, mesh=mesh, scratch_types=scratch)


_sc_agg = _make_sc_agg()
_sc_deg = _make_sc_deg()


def _tc_prep_body(x_ref, ew_ref, g_ref, b_ref, h_ref, ewl_ref):
  x = x_ref[...]
  mu = jnp.mean(x, axis=0, keepdims=True)
  var = jnp.mean((x - mu) ** 2, axis=0, keepdims=True)
  h_ref[...] = (x - mu) * lax.rsqrt(var + 1e-5) * g_ref[...] + b_ref[...]
  ewl_ref[...] = jnp.log1p(ew_ref[...])


def _tc_layer_body(h_ref, p_ref, deg_ref, g_ref, be_ref, ws_ref, wn_ref,
                   b_ref, out_ref, invdeg_ref):
  h = h_ref[...]
  agg = p_ref[0] + p_ref[1]
  deg = deg_ref[0] + deg_ref[1]
  inv = 1.0 / jnp.maximum(deg, 1.0)
  invdeg_ref[...] = inv
  z = (jnp.dot(h, ws_ref[...], preferred_element_type=jnp.float32)
       + jnp.dot(agg * inv, wn_ref[...], preferred_element_type=jnp.float32)
       + b_ref[...])
  z = jnp.maximum(z, 0.0)
  mu = jnp.mean(z, axis=0, keepdims=True)
  var = jnp.mean((z - mu) ** 2, axis=0, keepdims=True)
  out_ref[...] = (z - mu) * lax.rsqrt(var + 1e-5) * g_ref[...] + be_ref[...]


def _tc_final_body(h_ref, q_ref, invdeg_ref, ws_ref, wn_ref, b_ref,
                   lw_ref, lb_ref, out_ref):
  h = h_ref[...]
  neigh = (q_ref[0] + q_ref[1]) * invdeg_ref[...]
  z = (jnp.dot(h, ws_ref[...], preferred_element_type=jnp.float32)
       + jnp.dot(neigh, wn_ref[...], preferred_element_type=jnp.float32)
       + b_ref[...])
  z = jnp.maximum(z, 0.0)
  out_ref[...] = (jnp.dot(z, lw_ref[...], preferred_element_type=jnp.float32)
                  + lb_ref[...])


_tc_prep = pl.pallas_call(
    _tc_prep_body,
    out_shape=[jax.ShapeDtypeStruct((N, D), jnp.float32),
               jax.ShapeDtypeStruct((E // D, D), jnp.float32)],
)

_tc_layer = pl.pallas_call(
    _tc_layer_body,
    out_shape=[jax.ShapeDtypeStruct((N, D), jnp.float32),
               jax.ShapeDtypeStruct((N, 1), jnp.float32)],
)

_tc_final = pl.pallas_call(
    _tc_final_body,
    out_shape=jax.ShapeDtypeStruct((N, 1), jnp.float32),
)


def kernel(x, edge_index, edge_weight, bn0_gamma, bn0_beta, bn1_gamma,
           bn1_beta, W_self0, W_neigh0, b0, W_self1, W_neigh1, b1,
           lin_W, lin_b):
  src = edge_index[0]
  dst = edge_index[1]
  h0, ewl2d = _tc_prep(x, edge_weight.reshape(E // D, D),
                       bn0_gamma.reshape(1, D), bn0_beta.reshape(1, D))
  ewl = ewl2d.reshape(E)
  src3 = src.reshape(NW, SGN, SGC, CHUNK)
  dst3 = dst.reshape(NW, SGN, SGC, CHUNK)
  ewl3 = ewl.reshape(NW, SGN, SGC, CHUNK)
  (degp,) = _sc_deg(dst)
  (p_agg,) = _sc_agg(h0, src3, dst3, ewl3)
  h1n, invdeg = _tc_layer(h0, p_agg, degp[:, :, 0:1],
                          bn1_gamma.reshape(1, D),
                          bn1_beta.reshape(1, D), W_self0, W_neigh0,
                          b0.reshape(1, D))
  (q_agg,) = _sc_agg(h1n, src3, dst3, ewl3)
  out = _tc_final(h1n, q_agg, invdeg, W_self1, W_neigh1, b1.reshape(1, D),
                  lin_W, lin_b.reshape(1, 1))
  return out[:, 0]


# speedup vs baseline: 7.3536x; 1.1054x over previous
import functools

import jax
import jax.numpy as jnp
from jax import lax
from jax.experimental import pallas as pl
from jax.experimental.pallas import tpu as pltpu
from jax.experimental.pallas import tpu_sc as plsc

N = 10000
E = 320000
D = 128
NC = 2
NS = 16
LANES = 16
NW = NC * NS
EPW = E // NW
CHUNK = 80
NCHUNK = EPW // CHUNK
RPT = 624
RSPAN = 640


NBUF = 3
SGN = 5
SGC = NCHUNK // SGN


def _make_sc_agg():
  mesh = plsc.VectorSubcoreMesh(core_axis_name="c", subcore_axis_name="s")
  out_type = [jax.ShapeDtypeStruct((NC, N, D), jnp.float32)]
  scratch = [
      pltpu.VMEM((SGC, CHUNK), jnp.int32),
      pltpu.VMEM((SGC, CHUNK), jnp.int32),
      pltpu.VMEM((SGC, CHUNK), jnp.float32),
      [pltpu.VMEM((CHUNK, D), jnp.float32) for _ in range(NBUF)],
      [pltpu.SemaphoreType.DMA for _ in range(NBUF)],
      [pltpu.SemaphoreType.DMA for _ in range(NBUF)],
      pltpu.VMEM_SHARED((N, D), jnp.float32),
  ]

  def body(h, src4, dst4, ew4, out_agg, sidx, didx, ewv, rows, gsems,
           ssems, shagg):
    c = lax.axis_index("c")
    s = lax.axis_index("s")
    wid = s * NC + c
    row0 = s * RPT

    z16 = jnp.zeros((LANES,), jnp.float32)

    def zero_row(r, carry):
      for j in range(D // LANES):
        rows[0][r, pl.ds(j * LANES, LANES)] = z16
      return carry

    lax.fori_loop(0, CHUNK, zero_row, 0)
    for off in range(0, RSPAN, CHUNK):
      pltpu.sync_copy(rows[0], shagg.at[pl.ds(row0 + off, CHUNK)])
    plsc.subcore_barrier()

    def scale_chunk(i, buf):
      def scale_group(g, inner):
        w16 = ewv[i, pl.ds(g * LANES, LANES)]
        for k in range(LANES):
          w = lax.gather(
              w16, jnp.full((LANES, 1), k, jnp.int32),
              lax.GatherDimensionNumbers(offset_dims=(),
                                         collapsed_slice_dims=(0,),
                                         start_index_map=(0,)),
              (1,), mode=lax.GatherScatterMode.PROMISE_IN_BOUNDS)
          r = g * LANES + k
          for j in range(D // LANES):
            sl = pl.ds(j * LANES, LANES)
            buf[r, sl] = buf[r, sl] * w
        return inner

      lax.fori_loop(0, CHUNK // LANES, scale_group, 0)

    def super_body(sg, carry):
      pltpu.sync_copy(src4.at[wid, sg], sidx)
      pltpu.sync_copy(dst4.at[wid, sg], didx)
      pltpu.sync_copy(ew4.at[wid, sg], ewv)

      def group_body(p, inner):
        i0 = p * NBUF

        @pl.when(p > 0)
        def _drain_prev():
          for b in range(NBUF):
            pltpu.make_async_copy(rows[b], shagg.at[didx.at[0]],
                                  ssems[b]).wait()

        gd = []
        for b in range(NBUF):
          gd.append(pltpu.async_copy(h.at[sidx.at[i0 + b]], rows[b],
                                     gsems[b]))
        for b in range(NBUF):
          gd[b].wait()
          scale_chunk(i0 + b, rows[b])
          pltpu.async_copy(rows[b], shagg.at[didx.at[i0 + b]],
                           ssems[b], add=True)
        return inner

      lax.fori_loop(0, SGC // NBUF, group_body, 0)

      last = SGC - 1
      pltpu.make_async_copy(rows[0], shagg.at[didx.at[0]], ssems[0]).wait()
      pltpu.async_copy(h.at[sidx.at[last]], rows[0], gsems[0]).wait()
      scale_chunk(last, rows[0])
      pltpu.async_copy(rows[0], shagg.at[didx.at[last]], ssems[0],
                       add=True).wait()
      for b in range(1, NBUF):
        pltpu.make_async_copy(rows[b], shagg.at[didx.at[0]],
                              ssems[b]).wait()
      return carry

    lax.fori_loop(0, SGN, super_body, 0)

    plsc.subcore_barrier()
    pltpu.sync_copy(shagg.at[pl.ds(row0, RSPAN)],
                    out_agg.at[c, pl.ds(row0, RSPAN)])

  return pl.kernel(body, out_type, mesh=mesh, scratch_types=scratch)


def _make_sc_deg():
  mesh = plsc.VectorSubcoreMesh(core_axis_name="c", subcore_axis_name="s")
  out_type = [jax.ShapeDtypeStruct((NC, N, D), jnp.float32)]
  scratch = [
      pltpu.VMEM((SGC, CHUNK), jnp.int32),
      pltpu.VMEM((CHUNK, D), jnp.float32),
      pltpu.VMEM_SHARED((N, D), jnp.float32),
      pltpu.SemaphoreType.DMA,
  ]

  def body(dst4, out_deg, didx, ones, shdeg, sem):
    c = lax.axis_index("c")
    s = lax.axis_index("s")
    wid = s * NC + c
    row0 = s * RPT

    z16 = jnp.zeros((LANES,), jnp.float32)

    def zero_row(r, carry):
      for j in range(D // LANES):
        ones[r, pl.ds(j * LANES, LANES)] = z16
      return carry

    lax.fori_loop(0, CHUNK, zero_row, 0)
    for off in range(0, RSPAN, CHUNK):
      pltpu.sync_copy(ones, shdeg.at[pl.ds(row0 + off, CHUNK)])

    o16 = jnp.ones((LANES,), jnp.float32)

    def one_row(r, carry):
      for j in range(D // LANES):
        ones[r, pl.ds(j * LANES, LANES)] = o16
      return carry

    lax.fori_loop(0, CHUNK, one_row, 0)
    plsc.subcore_barrier()

    def super_body(sg, carry):
      pltpu.sync_copy(dst4.at[wid, sg], didx)

      def fire(i, inner):
        pltpu.async_copy(ones, shdeg.at[didx.at[i]], sem, add=True)
        return inner

      lax.fori_loop(0, SGC, fire, 0)

      def drain(i, inner):
        pltpu.make_async_copy(ones, shdeg.at[didx.at[0]], sem).wait()
        return inner

      lax.fori_loop(0, SGC, drain, 0)
      return carry

    lax.fori_loop(0, SGN, super_body, 0)
    plsc.subcore_barrier()

    pltpu.sync_copy(shdeg.at[pl.ds(row0, RSPAN)],
                    out_deg.at[c, pl.ds(row0, RSPAN)])

  return pl.kernel(body, out_type, mesh=mesh, scratch_types=scratch)


_sc_agg = _make_sc_agg()
_sc_deg = _make_sc_deg()


def _tc_prep_body(x_ref, ew_ref, g_ref, b_ref, h_ref, ewl_ref):
  x = x_ref[...]
  mu = jnp.mean(x, axis=0, keepdims=True)
  var = jnp.mean((x - mu) ** 2, axis=0, keepdims=True)
  h_ref[...] = (x - mu) * lax.rsqrt(var + 1e-5) * g_ref[...] + b_ref[...]
  ewl_ref[...] = jnp.log1p(ew_ref[...])


def _tc_layer_body(h_ref, p_ref, deg_ref, g_ref, be_ref, ws_ref, wn_ref,
                   b_ref, out_ref, invdeg_ref):
  h = h_ref[...]
  agg = p_ref[0] + p_ref[1]
  deg = deg_ref[0] + deg_ref[1]
  inv = 1.0 / jnp.maximum(deg, 1.0)
  invdeg_ref[...] = inv
  z = (jnp.dot(h, ws_ref[...], preferred_element_type=jnp.float32)
       + jnp.dot(agg * inv, wn_ref[...], preferred_element_type=jnp.float32)
       + b_ref[...])
  z = jnp.maximum(z, 0.0)
  mu = jnp.mean(z, axis=0, keepdims=True)
  var = jnp.mean((z - mu) ** 2, axis=0, keepdims=True)
  out_ref[...] = (z - mu) * lax.rsqrt(var + 1e-5) * g_ref[...] + be_ref[...]


def _tc_final_body(h_ref, q_ref, invdeg_ref, ws_ref, wn_ref, b_ref,
                   lw_ref, lb_ref, out_ref):
  h = h_ref[...]
  neigh = (q_ref[0] + q_ref[1]) * invdeg_ref[...]
  z = (jnp.dot(h, ws_ref[...], preferred_element_type=jnp.float32)
       + jnp.dot(neigh, wn_ref[...], preferred_element_type=jnp.float32)
       + b_ref[...])
  z = jnp.maximum(z, 0.0)
  out_ref[...] = (jnp.dot(z, lw_ref[...], preferred_element_type=jnp.float32)
                  + lb_ref[...])


_tc_prep = pl.pallas_call(
    _tc_prep_body,
    out_shape=[jax.ShapeDtypeStruct((N, D), jnp.float32),
               jax.ShapeDtypeStruct((E // D, D), jnp.float32)],
)

_tc_layer = pl.pallas_call(
    _tc_layer_body,
    out_shape=[jax.ShapeDtypeStruct((N, D), jnp.float32),
               jax.ShapeDtypeStruct((N, 1), jnp.float32)],
)

_tc_final = pl.pallas_call(
    _tc_final_body,
    out_shape=jax.ShapeDtypeStruct((N, 1), jnp.float32),
)


def kernel(x, edge_index, edge_weight, bn0_gamma, bn0_beta, bn1_gamma,
           bn1_beta, W_self0, W_neigh0, b0, W_self1, W_neigh1, b1,
           lin_W, lin_b):
  src = edge_index[0]
  dst = edge_index[1]
  h0, ewl2d = _tc_prep(x, edge_weight.reshape(E // D, D),
                       bn0_gamma.reshape(1, D), bn0_beta.reshape(1, D))
  ewl = ewl2d.reshape(E)
  src3 = src.reshape(NW, SGN, SGC, CHUNK)
  dst3 = dst.reshape(NW, SGN, SGC, CHUNK)
  ewl3 = ewl.reshape(NW, SGN, SGC, CHUNK)
  (degp,) = _sc_deg(dst3)
  (p_agg,) = _sc_agg(h0, src3, dst3, ewl3)
  h1n, invdeg = _tc_layer(h0, p_agg, degp[:, :, 0:1],
                          bn1_gamma.reshape(1, D),
                          bn1_beta.reshape(1, D), W_self0, W_neigh0,
                          b0.reshape(1, D))
  (q_agg,) = _sc_agg(h1n, src3, dst3, ewl3)
  out = _tc_final(h1n, q_agg, invdeg, W_self1, W_neigh1, b1.reshape(1, D),
                  lin_W, lin_b.reshape(1, 1))
  return out[:, 0]
